# baseline (speedup 1.0000x reference)
import functools

import jax
import jax.numpy as jnp
from jax import lax
from jax.experimental import pallas as pl
from jax.experimental.pallas import tpu as pltpu
from jax.experimental.pallas import tpu_sc as plsc

N = 10000
E = 320000
D_EMB = 128
HID = 256
NCLS = 64

NC, NS = 2, 16
NW = NC * NS
CHUNK = 125
EROWS = E // CHUNK
ACC = 10240
SROW = ACC // NS
SLAB = 64

_MESH = plsc.VectorSubcoreMesh(core_axis_name="c", subcore_axis_name="s")
_CP = pltpu.CompilerParams(use_tc_tiling_on_sc=False)


def _deg_body(dstr, degp, ddst, e0, zb, degacc, dsem):
    c = lax.axis_index("c")
    s = lax.axis_index("s")
    w = c * NS + s
    rpw = EROWS // NW
    zero16 = jnp.zeros((16,), jnp.float32)

    def zrow(i, _):
        zb[i, :] = zero16
        return 0
    lax.fori_loop(0, SROW, zrow, 0)
    pltpu.sync_copy(zb, degacc.at[pl.ds(s * SROW, SROW)])

    lane = lax.broadcasted_iota(jnp.int32, (16,), 0)
    one0 = jnp.where(lane == 0, 1.0, 0.0).astype(jnp.float32)

    def erow(i, _):
        e0[i, :] = one0
        return 0
    lax.fori_loop(0, CHUNK, erow, 0)

    pltpu.sync_copy(dstr.at[pl.ds(w * rpw, rpw), :], ddst)
    plsc.subcore_barrier()

    DEGW = 8

    def issue(i):
        pltpu.async_copy(e0, degacc.at[ddst.at[i]], dsem, add=True)

    def retire(i):
        pltpu.make_async_copy(e0, degacc.at[ddst.at[i]], dsem).wait()

    for i in range(DEGW):
        issue(i)

    def drow(i, _):
        retire(i - DEGW)
        issue(i)
        return 0
    lax.fori_loop(DEGW, rpw, drow, 0)

    def drain(i, _):
        retire(i)
        return 0
    lax.fori_loop(rpw - DEGW, rpw, drain, 0)

    plsc.subcore_barrier()
    pltpu.sync_copy(degacc.at[pl.ds(s * SROW, SROW)],
                    degp.at[c, pl.ds(s * SROW, SROW)])


_deg = pl.kernel(
    _deg_body,
    out_type=jax.ShapeDtypeStruct((NC, ACC, 16), jnp.float32),
    mesh=_MESH,
    compiler_params=_CP,
    scratch_types=[
        pltpu.VMEM((EROWS // NW, CHUNK), jnp.int32),
        pltpu.VMEM((CHUNK, 16), jnp.float32),
        pltpu.VMEM((SROW, 16), jnp.float32),
        pltpu.VMEM_SHARED((ACC, 16), jnp.float32),
        pltpu.SemaphoreType.DMA,
    ],
)


RING = 5


def _spmm_run(xin, nrows, sidx, didx, bufs, acc, sems):
    def gath(j, b):
        pltpu.async_copy(xin.at[sidx.at[j]], bufs[b], sems[b])

    def wait_gath(j, b):
        pltpu.make_async_copy(xin.at[sidx.at[j]], bufs[b], sems[b]).wait()

    def scat(j, b):
        pltpu.async_copy(bufs[b], acc.at[didx.at[j]], sems[b], add=True)

    def wait_scat(j, b):
        pltpu.make_async_copy(bufs[b], acc.at[didx.at[j]], sems[b]).wait()

    gath(0, 0)
    gath(1, 1)
    gath(2, 2); wait_gath(0, 0); scat(0, 0)
    gath(3, 3); wait_gath(1, 1); scat(1, 1)
    gath(4, 4); wait_gath(2, 2); scat(2, 2)
    wait_scat(0, 0); gath(5, 0); wait_gath(3, 3); scat(3, 3)
    wait_scat(1, 1); gath(6, 1); wait_gath(4, 4); scat(4, 4)

    def group(g, _):
        j0 = g * RING
        for b in range(RING):
            j = j0 + b
            bg = (b + 2) % RING
            wait_scat(j - 3, bg)

            @pl.when(j + 2 < nrows)
            def _(j=j, bg=bg):
                gath(j + 2, bg)
            wait_gath(j, b)
            scat(j, b)
        return 0
    lax.fori_loop(1, nrows // RING, group, 0)
    wait_scat(nrows - 3, 2)
    wait_scat(nrows - 2, 3)
    wait_scat(nrows - 1, 4)


def _zero_stripe(buf0, acc, s):
    zero16 = jnp.zeros((16,), jnp.float32)

    def zrow(i, _):
        for k in range(SLAB // 16):
            buf0[i, pl.ds(k * 16, 16)] = zero16
        return 0
    lax.fori_loop(0, CHUNK, zrow, 0)
    for r in range(SROW // CHUNK):
        pltpu.sync_copy(buf0, acc.at[pl.ds(s * SROW + r * CHUNK, CHUNK)])
    pltpu.sync_copy(buf0.at[pl.ds(0, SROW - (SROW // CHUNK) * CHUNK)],
                    acc.at[pl.ds(s * SROW + (SROW // CHUNK) * CHUNK,
                                 SROW - (SROW // CHUNK) * CHUNK)])


def _spmm_feat_body(*refs, P):
    xs = refs[:2 * P]
    srcr, dstr, sout = refs[2 * P], refs[2 * P + 1], refs[2 * P + 2]
    sidx, didx = refs[2 * P + 3], refs[2 * P + 4]
    bufs = refs[2 * P + 5:2 * P + 10]
    acc = refs[2 * P + 10]
    sems = refs[2 * P + 11:2 * P + 16]

    c = lax.axis_index("c")
    s = lax.axis_index("s")
    nrows = EROWS // NS
    base = s * nrows

    pltpu.sync_copy(srcr.at[pl.ds(base, nrows), :], sidx)
    pltpu.sync_copy(dstr.at[pl.ds(base, nrows), :], didx)

    for p in range(P):
        _zero_stripe(bufs[0], acc, s)
        plsc.subcore_barrier()

        @pl.when(c == 0)
        def _(p=p):
            _spmm_run(xs[p], nrows, sidx, didx, bufs, acc, sems)

        @pl.when(c == 1)
        def _(p=p):
            _spmm_run(xs[P + p], nrows, sidx, didx, bufs, acc, sems)

        plsc.subcore_barrier()
        pltpu.sync_copy(acc.at[pl.ds(s * SROW, SROW)],
                        sout.at[c, p, pl.ds(s * SROW, SROW)])
        if p + 1 < P:
            plsc.subcore_barrier()


def _spmm_edge_body(xin, srcr, dstr, sout,
                    sidx, didx, b0, b1, b2, b3, b4, acc,
                    m0, m1, m2, m3, m4):
    c = lax.axis_index("c")
    s = lax.axis_index("s")
    nrows = EROWS // NW
    base = (c * NS + s) * nrows
    bufs = (b0, b1, b2, b3, b4)
    sems = (m0, m1, m2, m3, m4)

    pltpu.sync_copy(srcr.at[pl.ds(base, nrows), :], sidx)
    pltpu.sync_copy(dstr.at[pl.ds(base, nrows), :], didx)

    _zero_stripe(b0, acc, s)
    plsc.subcore_barrier()
    _spmm_run(xin, nrows, sidx, didx, bufs, acc, sems)
    plsc.subcore_barrier()
    pltpu.sync_copy(acc.at[pl.ds(s * SROW, SROW)],
                    sout.at[c, pl.ds(s * SROW, SROW)])


def _make_spmm_feat(P):
    nrows = EROWS // NS
    return pl.kernel(
        functools.partial(_spmm_feat_body, P=P),
        out_type=jax.ShapeDtypeStruct((NC, P, ACC, SLAB), jnp.float32),
        mesh=_MESH,
        compiler_params=_CP,
        scratch_types=(
            [pltpu.VMEM((nrows, CHUNK), jnp.int32)] * 2
            + [pltpu.VMEM((CHUNK, SLAB), jnp.float32)] * RING
            + [pltpu.VMEM_SHARED((ACC, SLAB), jnp.float32)]
            + [pltpu.SemaphoreType.DMA] * RING
        ),
    )


_spmm1 = _make_spmm_feat(1)
_spmm2 = _make_spmm_feat(2)

_spmm3 = pl.kernel(
    _spmm_edge_body,
    out_type=jax.ShapeDtypeStruct((NC, ACC, SLAB), jnp.float32),
    mesh=_MESH,
    compiler_params=_CP,
    scratch_types=(
        [pltpu.VMEM((EROWS // NW, CHUNK), jnp.int32)] * 2
        + [pltpu.VMEM((CHUNK, SLAB), jnp.float32)] * RING
        + [pltpu.VMEM_SHARED((ACC, SLAB), jnp.float32)]
        + [pltpu.SemaphoreType.DMA] * RING
    ),
)


def _gather_body(out3, nodes, res, idxv, gbuf, sem):
    c = lax.axis_index("c")
    s = lax.axis_index("s")
    w = s * NC + c
    base = jnp.minimum(w * 320, N - 320)
    pltpu.sync_copy(nodes.at[pl.ds(base, 320)], idxv)
    for lo, sz in ((0, 128), (128, 128), (256, 64)):
        pltpu.async_copy(out3.at[idxv.at[pl.ds(lo, sz)]],
                         gbuf.at[pl.ds(lo, sz)], sem)
    for lo, sz in ((0, 128), (128, 128), (256, 64)):
        pltpu.make_async_copy(out3.at[idxv.at[pl.ds(lo, sz)]],
                              gbuf.at[pl.ds(lo, sz)], sem).wait()
    pltpu.sync_copy(gbuf, res.at[pl.ds(base, 320)])


_gather = pl.kernel(
    _gather_body,
    out_type=jax.ShapeDtypeStruct((N, NCLS), jnp.float32),
    mesh=_MESH,
    compiler_params=_CP,
    scratch_types=[
        pltpu.VMEM((320,), jnp.int32),
        pltpu.VMEM((320, NCLS), jnp.float32),
        pltpu.SemaphoreType.DMA,
    ],
)


_BR = 1000


def _tca_body(deg_ref, emb_ref, dinv_ref, xpa_ref, xpb_ref):
    d = deg_ref[0, :, 0:1] + deg_ref[1, :, 0:1] + 1.0
    dinv = lax.rsqrt(d)
    dinv_ref[...] = dinv
    xp = emb_ref[...] * dinv
    xpa_ref[...] = xp[:, :SLAB]
    xpb_ref[...] = xp[:, SLAB:]


def _tca(degp, emb):
    return pl.pallas_call(
        _tca_body,
        grid=(N // _BR,),
        in_specs=[pl.BlockSpec((NC, _BR, 16), lambda i: (0, i, 0)),
                  pl.BlockSpec((_BR, D_EMB), lambda i: (i, 0))],
        out_specs=(pl.BlockSpec((_BR, 1), lambda i: (i, 0)),
                   pl.BlockSpec((_BR, SLAB), lambda i: (i, 0)),
                   pl.BlockSpec((_BR, SLAB), lambda i: (i, 0))),
        out_shape=(jax.ShapeDtypeStruct((N, 1), jnp.float32),
                   jax.ShapeDtypeStruct((N, SLAB), jnp.float32),
                   jax.ShapeDtypeStruct((N, SLAB), jnp.float32)),
    )(degp, emb)


def _layer1_body(s_ref, xpa_ref, xpb_ref, dinv_ref, w_ref, b_ref,
                 o0_ref, o1_ref, o2_ref, o3_ref):
    dinv = dinv_ref[...]
    sagg = jnp.concatenate([s_ref[0, 0], s_ref[1, 0]], axis=1)
    xp = jnp.concatenate([xpa_ref[...], xpb_ref[...]], axis=1)
    t = (sagg + xp) * dinv
    y = jnp.dot(t, w_ref[...], preferred_element_type=jnp.float32)
    h = jax.nn.relu(y + b_ref[...]) * dinv
    o0_ref[...] = h[:, 0 * SLAB:1 * SLAB]
    o1_ref[...] = h[:, 1 * SLAB:2 * SLAB]
    o2_ref[...] = h[:, 2 * SLAB:3 * SLAB]
    o3_ref[...] = h[:, 3 * SLAB:4 * SLAB]


def _tc1(s1, xpa, xpb, dinv, W1, b1):
    slabspec = pl.BlockSpec((_BR, SLAB), lambda i: (i, 0))
    return pl.pallas_call(
        _layer1_body,
        grid=(N // _BR,),
        in_specs=[pl.BlockSpec((NC, 1, _BR, SLAB), lambda i: (0, 0, i, 0)),
                  slabspec, slabspec,
                  pl.BlockSpec((_BR, 1), lambda i: (i, 0)),
                  pl.BlockSpec((D_EMB, HID), lambda i: (0, 0)),
                  pl.BlockSpec((1, HID), lambda i: (0, 0))],
        out_specs=(slabspec, slabspec, slabspec, slabspec),
        out_shape=tuple(jax.ShapeDtypeStruct((N, SLAB), jnp.float32)
                        for _ in range(4)),
    )(s1, xpa, xpb, dinv, W1, b1)


def _layer2_body(s_ref, h0_ref, h1_ref, h2_ref, h3_ref, dinv_ref,
                 w2_ref, b2_ref, w3_ref, o_ref):
    dinv = dinv_ref[...]
    sagg = jnp.concatenate(
        [s_ref[0, 0], s_ref[0, 1], s_ref[1, 0], s_ref[1, 1]], axis=1)
    hfull = jnp.concatenate(
        [h0_ref[...], h1_ref[...], h2_ref[...], h3_ref[...]], axis=1)
    t = (sagg + hfull) * dinv
    y = jnp.dot(t, w2_ref[...], preferred_element_type=jnp.float32)
    h2p = jax.nn.relu(y + b2_ref[...]) * dinv
    o_ref[...] = jnp.dot(h2p, w3_ref[...], preferred_element_type=jnp.float32)


def _tc2(s2, h0, h1, h2, h3, dinv, W2, b2, W3):
    slabspec = pl.BlockSpec((_BR, SLAB), lambda i: (i, 0))
    return pl.pallas_call(
        _layer2_body,
        grid=(N // _BR,),
        in_specs=[pl.BlockSpec((NC, 2, _BR, SLAB), lambda i: (0, 0, i, 0)),
                  slabspec, slabspec, slabspec, slabspec,
                  pl.BlockSpec((_BR, 1), lambda i: (i, 0)),
                  pl.BlockSpec((HID, HID), lambda i: (0, 0)),
                  pl.BlockSpec((1, HID), lambda i: (0, 0)),
                  pl.BlockSpec((HID, NCLS), lambda i: (0, 0))],
        out_specs=pl.BlockSpec((_BR, NCLS), lambda i: (i, 0)),
        out_shape=jax.ShapeDtypeStruct((N, NCLS), jnp.float32),
    )(s2, h0, h1, h2, h3, dinv, W2, b2, W3)


def _layer3_body(s_ref, g_ref, dinv_ref, b_ref, o_ref):
    o_ref[...] = ((s_ref[0] + s_ref[1] + g_ref[...]) * dinv_ref[...]
                  + b_ref[...])


def _tc3(s3, gp, dinv, b3):
    return pl.pallas_call(
        _layer3_body,
        grid=(N // _BR,),
        in_specs=[pl.BlockSpec((NC, _BR, NCLS), lambda i: (0, i, 0)),
                  pl.BlockSpec((_BR, NCLS), lambda i: (i, 0)),
                  pl.BlockSpec((_BR, 1), lambda i: (i, 0)),
                  pl.BlockSpec((1, NCLS), lambda i: (0, 0))],
        out_specs=pl.BlockSpec((_BR, NCLS), lambda i: (i, 0)),
        out_shape=jax.ShapeDtypeStruct((N, NCLS), jnp.float32),
    )(s3, gp, dinv, b3)


def kernel(nodes, edge_index, emb_weight, W1, b1, W2, b2, W3, b3):
    srcr = edge_index[0].reshape(EROWS, CHUNK)
    dstr = edge_index[1].reshape(EROWS, CHUNK)

    degp = _deg(dstr)
    dinv, xpa, xpb = _tca(degp, emb_weight)
    s1 = _spmm1(xpa, xpb, srcr, dstr)
    h0, h1, h2, h3 = _tc1(s1, xpa, xpb, dinv, W1, b1.reshape(1, HID))
    s2 = _spmm2(h0, h1, h2, h3, srcr, dstr)
    gp = _tc2(s2, h0, h1, h2, h3, dinv, W2, b2.reshape(1, HID), W3)
    s3 = _spmm3(gp, srcr, dstr)
    out3 = _tc3(s3, gp, dinv, b3.reshape(1, NCLS))
    return _gather(out3, nodes)


# speedup vs baseline: 1.0636x; 1.0636x over previous
import functools

import jax
import jax.numpy as jnp
from jax import lax
from jax.experimental import pallas as pl
from jax.experimental.pallas import tpu as pltpu
from jax.experimental.pallas import tpu_sc as plsc

N = 10000
E = 320000
D_EMB = 128
HID = 256
NCLS = 64

NC, NS = 2, 16
NW = NC * NS
CHUNK = 125
EROWS = E // CHUNK
ACC = 10240
SROW = ACC // NS

_MESH = plsc.VectorSubcoreMesh(core_axis_name="c", subcore_axis_name="s")
_CP = pltpu.CompilerParams(use_tc_tiling_on_sc=False)


def _deg_body(dstr, degp, ddst, e0, zb, degacc, dsem):
    c = lax.axis_index("c")
    s = lax.axis_index("s")
    w = c * NS + s
    rpw = EROWS // NW
    zero16 = jnp.zeros((16,), jnp.float32)

    def zrow(i, _):
        zb[i, :] = zero16
        return 0
    lax.fori_loop(0, SROW, zrow, 0)
    pltpu.sync_copy(zb, degacc.at[pl.ds(s * SROW, SROW)])

    lane = lax.broadcasted_iota(jnp.int32, (16,), 0)
    one0 = jnp.where(lane == 0, 1.0, 0.0).astype(jnp.float32)

    def erow(i, _):
        e0[i, :] = one0
        return 0
    lax.fori_loop(0, CHUNK, erow, 0)

    pltpu.sync_copy(dstr.at[pl.ds(w * rpw, rpw), :], ddst)
    plsc.subcore_barrier()

    DEGW = 8

    def issue(i):
        pltpu.async_copy(e0, degacc.at[ddst.at[i]], dsem, add=True)

    def retire(i):
        pltpu.make_async_copy(e0, degacc.at[ddst.at[i]], dsem).wait()

    for i in range(DEGW):
        issue(i)

    def drow(i, _):
        retire(i - DEGW)
        issue(i)
        return 0
    lax.fori_loop(DEGW, rpw, drow, 0)

    def drain(i, _):
        retire(i)
        return 0
    lax.fori_loop(rpw - DEGW, rpw, drain, 0)

    plsc.subcore_barrier()
    pltpu.sync_copy(degacc.at[pl.ds(s * SROW, SROW)],
                    degp.at[c, pl.ds(s * SROW, SROW)])


_deg = pl.kernel(
    _deg_body,
    out_type=jax.ShapeDtypeStruct((NC, ACC, 16), jnp.float32),
    mesh=_MESH,
    compiler_params=_CP,
    scratch_types=[
        pltpu.VMEM((EROWS // NW, CHUNK), jnp.int32),
        pltpu.VMEM((CHUNK, 16), jnp.float32),
        pltpu.VMEM((SROW, 16), jnp.float32),
        pltpu.VMEM_SHARED((ACC, 16), jnp.float32),
        pltpu.SemaphoreType.DMA,
    ],
)


SEG = 40


def _spmm_run(xin, srcr, dstr, base, nseg,
              sseg, dseg, buf0, buf1, acc, sem0, sem1):
    for g in range(nseg):
        pltpu.sync_copy(srcr.at[pl.ds(base + g * SEG, SEG), :], sseg)
        pltpu.sync_copy(dstr.at[pl.ds(base + g * SEG, SEG), :], dseg)
        pltpu.async_copy(xin.at[sseg.at[0]], buf0, sem0)

        def pbody(p, _):
            j0 = 2 * p
            pltpu.async_copy(xin.at[sseg.at[j0 + 1]], buf1, sem1)
            pltpu.make_async_copy(xin.at[sseg.at[j0]], buf0, sem0).wait()
            pltpu.sync_copy(buf0, acc.at[dseg.at[j0]], add=True)

            @pl.when(p + 1 < SEG // 2)
            def _():
                pltpu.async_copy(xin.at[sseg.at[j0 + 2]], buf0, sem0)
            pltpu.make_async_copy(xin.at[sseg.at[j0 + 1]], buf1, sem1).wait()
            pltpu.sync_copy(buf1, acc.at[dseg.at[j0 + 1]], add=True)
            return 0
        lax.fori_loop(0, SEG // 2, pbody, 0)


def _spmm_body(xa, xb, srcr, dstr, sout,
               sseg, dseg, buf0, buf1, acc, sem0, sem1, *, F, split):
    c = lax.axis_index("c")
    s = lax.axis_index("s")

    if split == "edge":
        nrows = EROWS // NW
        base = (c * NS + s) * nrows
    else:
        nrows = EROWS // NS
        base = s * nrows
    nseg = nrows // SEG

    zero16 = jnp.zeros((16,), jnp.float32)

    def zrow(i, _):
        for k in range(F // 16):
            buf0[i, pl.ds(k * 16, 16)] = zero16
        return 0
    lax.fori_loop(0, CHUNK, zrow, 0)
    for r in range(SROW // CHUNK):
        pltpu.sync_copy(buf0, acc.at[pl.ds(s * SROW + r * CHUNK, CHUNK)])
    pltpu.sync_copy(buf0.at[pl.ds(0, SROW - (SROW // CHUNK) * CHUNK)],
                    acc.at[pl.ds(s * SROW + (SROW // CHUNK) * CHUNK,
                                 SROW - (SROW // CHUNK) * CHUNK)])
    plsc.subcore_barrier()

    if split == "edge":
        _spmm_run(xa, srcr, dstr, base, nseg,
                  sseg, dseg, buf0, buf1, acc, sem0, sem1)
    else:
        @pl.when(c == 0)
        def _():
            _spmm_run(xa, srcr, dstr, base, nseg,
                      sseg, dseg, buf0, buf1, acc, sem0, sem1)

        @pl.when(c == 1)
        def _():
            _spmm_run(xb, srcr, dstr, base, nseg,
                      sseg, dseg, buf0, buf1, acc, sem0, sem1)

    plsc.subcore_barrier()
    pltpu.sync_copy(acc.at[pl.ds(s * SROW, SROW)],
                    sout.at[c, pl.ds(s * SROW, SROW)])


def _make_spmm(F, split):
    return pl.kernel(
        functools.partial(_spmm_body, F=F, split=split),
        out_type=jax.ShapeDtypeStruct((NC, ACC, F), jnp.float32),
        mesh=_MESH,
        compiler_params=_CP,
        scratch_types=[
            pltpu.VMEM((SEG, CHUNK), jnp.int32),
            pltpu.VMEM((SEG, CHUNK), jnp.int32),
            pltpu.VMEM((CHUNK, F), jnp.float32),
            pltpu.VMEM((CHUNK, F), jnp.float32),
            pltpu.VMEM_SHARED((ACC, F), jnp.float32),
            pltpu.SemaphoreType.DMA,
            pltpu.SemaphoreType.DMA,
        ],
    )


_spmm1 = _make_spmm(D_EMB, "edge")
_spmm2 = _make_spmm(HID // 2, "feat")


RING = 5


def _ring_run(xin, nrows, sidx, didx, bufs, acc, sems):
    def gath(j, b):
        pltpu.async_copy(xin.at[sidx.at[j]], bufs[b], sems[b])

    def wait_gath(j, b):
        pltpu.make_async_copy(xin.at[sidx.at[j]], bufs[b], sems[b]).wait()

    def scat(j, b):
        pltpu.async_copy(bufs[b], acc.at[didx.at[j]], sems[b], add=True)

    def wait_scat(j, b):
        pltpu.make_async_copy(bufs[b], acc.at[didx.at[j]], sems[b]).wait()

    gath(0, 0)
    gath(1, 1)
    gath(2, 2); wait_gath(0, 0); scat(0, 0)
    gath(3, 3); wait_gath(1, 1); scat(1, 1)
    gath(4, 4); wait_gath(2, 2); scat(2, 2)
    wait_scat(0, 0); gath(5, 0); wait_gath(3, 3); scat(3, 3)
    wait_scat(1, 1); gath(6, 1); wait_gath(4, 4); scat(4, 4)

    def group(g, _):
        j0 = g * RING
        for b in range(RING):
            j = j0 + b
            bg = (b + 2) % RING
            wait_scat(j - 3, bg)

            @pl.when(j + 2 < nrows)
            def _(j=j, bg=bg):
                gath(j + 2, bg)
            wait_gath(j, b)
            scat(j, b)
        return 0
    lax.fori_loop(1, nrows // RING, group, 0)
    wait_scat(nrows - 3, 2)
    wait_scat(nrows - 2, 3)
    wait_scat(nrows - 1, 4)


def _spmm3_body(xin, srcr, dstr, sout,
                sidx, didx, b0, b1, b2, b3, b4, acc,
                m0, m1, m2, m3, m4):
    c = lax.axis_index("c")
    s = lax.axis_index("s")
    nrows = EROWS // NW
    base = (c * NS + s) * nrows
    bufs = (b0, b1, b2, b3, b4)
    sems = (m0, m1, m2, m3, m4)

    pltpu.sync_copy(srcr.at[pl.ds(base, nrows), :], sidx)
    pltpu.sync_copy(dstr.at[pl.ds(base, nrows), :], didx)

    zero16 = jnp.zeros((16,), jnp.float32)

    def zrow(i, _):
        for k in range(NCLS // 16):
            b0[i, pl.ds(k * 16, 16)] = zero16
        return 0
    lax.fori_loop(0, CHUNK, zrow, 0)
    for r in range(SROW // CHUNK):
        pltpu.sync_copy(b0, acc.at[pl.ds(s * SROW + r * CHUNK, CHUNK)])
    pltpu.sync_copy(b0.at[pl.ds(0, SROW - (SROW // CHUNK) * CHUNK)],
                    acc.at[pl.ds(s * SROW + (SROW // CHUNK) * CHUNK,
                                 SROW - (SROW // CHUNK) * CHUNK)])
    plsc.subcore_barrier()
    _ring_run(xin, nrows, sidx, didx, bufs, acc, sems)
    plsc.subcore_barrier()
    pltpu.sync_copy(acc.at[pl.ds(s * SROW, SROW)],
                    sout.at[c, pl.ds(s * SROW, SROW)])


_spmm3 = pl.kernel(
    _spmm3_body,
    out_type=jax.ShapeDtypeStruct((NC, ACC, NCLS), jnp.float32),
    mesh=_MESH,
    compiler_params=_CP,
    scratch_types=(
        [pltpu.VMEM((EROWS // NW, CHUNK), jnp.int32)] * 2
        + [pltpu.VMEM((CHUNK, NCLS), jnp.float32)] * RING
        + [pltpu.VMEM_SHARED((ACC, NCLS), jnp.float32)]
        + [pltpu.SemaphoreType.DMA] * RING
    ),
)


def _gather_body(out3, nodes, res, idxv, gbuf, sem):
    c = lax.axis_index("c")
    s = lax.axis_index("s")
    w = s * NC + c
    base = jnp.minimum(w * 320, N - 320)
    pltpu.sync_copy(nodes.at[pl.ds(base, 320)], idxv)
    for lo, sz in ((0, 128), (128, 128), (256, 64)):
        pltpu.async_copy(out3.at[idxv.at[pl.ds(lo, sz)]],
                         gbuf.at[pl.ds(lo, sz)], sem)
    for lo, sz in ((0, 128), (128, 128), (256, 64)):
        pltpu.make_async_copy(out3.at[idxv.at[pl.ds(lo, sz)]],
                              gbuf.at[pl.ds(lo, sz)], sem).wait()
    pltpu.sync_copy(gbuf, res.at[pl.ds(base, 320)])


_gather = pl.kernel(
    _gather_body,
    out_type=jax.ShapeDtypeStruct((N, NCLS), jnp.float32),
    mesh=_MESH,
    compiler_params=_CP,
    scratch_types=[
        pltpu.VMEM((320,), jnp.int32),
        pltpu.VMEM((320, NCLS), jnp.float32),
        pltpu.SemaphoreType.DMA,
    ],
)


_BR = 1000


def _row_spec(F):
    return pl.BlockSpec((NC, _BR, F), lambda i: (0, i, 0))


def _tca_body(deg_ref, emb_ref, dinv_ref, xp_ref):
    d = deg_ref[0, :, 0:1] + deg_ref[1, :, 0:1] + 1.0
    dinv = lax.rsqrt(d)
    dinv_ref[...] = dinv
    xp_ref[...] = emb_ref[...] * dinv


def _tca(degp, emb):
    return pl.pallas_call(
        _tca_body,
        grid=(N // _BR,),
        in_specs=[pl.BlockSpec((NC, _BR, 16), lambda i: (0, i, 0)),
                  pl.BlockSpec((_BR, D_EMB), lambda i: (i, 0))],
        out_specs=(pl.BlockSpec((_BR, 1), lambda i: (i, 0)),
                   pl.BlockSpec((_BR, D_EMB), lambda i: (i, 0))),
        out_shape=(jax.ShapeDtypeStruct((N, 1), jnp.float32),
                   jax.ShapeDtypeStruct((N, D_EMB), jnp.float32)),
    )(degp, emb)


def _layer1_body(s_ref, xp_ref, dinv_ref, w_ref, b_ref, oa_ref, ob_ref):
    dinv = dinv_ref[...]
    t = (s_ref[0] + s_ref[1] + xp_ref[...]) * dinv
    y = jnp.dot(t, w_ref[...], preferred_element_type=jnp.float32)
    h = jax.nn.relu(y + b_ref[...]) * dinv
    oa_ref[...] = h[:, : HID // 2]
    ob_ref[...] = h[:, HID // 2:]


def _tc1(s1, xp, dinv, W1, b1):
    return pl.pallas_call(
        _layer1_body,
        grid=(N // _BR,),
        in_specs=[_row_spec(D_EMB),
                  pl.BlockSpec((_BR, D_EMB), lambda i: (i, 0)),
                  pl.BlockSpec((_BR, 1), lambda i: (i, 0)),
                  pl.BlockSpec((D_EMB, HID), lambda i: (0, 0)),
                  pl.BlockSpec((1, HID), lambda i: (0, 0))],
        out_specs=(pl.BlockSpec((_BR, HID // 2), lambda i: (i, 0)),
                   pl.BlockSpec((_BR, HID // 2), lambda i: (i, 0))),
        out_shape=(jax.ShapeDtypeStruct((N, HID // 2), jnp.float32),
                   jax.ShapeDtypeStruct((N, HID // 2), jnp.float32)),
    )(s1, xp, dinv, W1, b1)


def _layer2_body(s_ref, ha_ref, hb_ref, dinv_ref, w2_ref, b2_ref, w3_ref,
                 o_ref):
    dinv = dinv_ref[...]
    sfull = jnp.concatenate([s_ref[0], s_ref[1]], axis=1)
    hfull = jnp.concatenate([ha_ref[...], hb_ref[...]], axis=1)
    t = (sfull + hfull) * dinv
    y = jnp.dot(t, w2_ref[...], preferred_element_type=jnp.float32)
    h2p = jax.nn.relu(y + b2_ref[...]) * dinv
    o_ref[...] = jnp.dot(h2p, w3_ref[...], preferred_element_type=jnp.float32)


def _tc2(s2, h1a, h1b, dinv, W2, b2, W3):
    return pl.pallas_call(
        _layer2_body,
        grid=(N // _BR,),
        in_specs=[_row_spec(HID // 2),
                  pl.BlockSpec((_BR, HID // 2), lambda i: (i, 0)),
                  pl.BlockSpec((_BR, HID // 2), lambda i: (i, 0)),
                  pl.BlockSpec((_BR, 1), lambda i: (i, 0)),
                  pl.BlockSpec((HID, HID), lambda i: (0, 0)),
                  pl.BlockSpec((1, HID), lambda i: (0, 0)),
                  pl.BlockSpec((HID, NCLS), lambda i: (0, 0))],
        out_specs=pl.BlockSpec((_BR, NCLS), lambda i: (i, 0)),
        out_shape=jax.ShapeDtypeStruct((N, NCLS), jnp.float32),
    )(s2, h1a, h1b, dinv, W2, b2, W3)


def _layer3_body(s_ref, g_ref, dinv_ref, b_ref, o_ref):
    o_ref[...] = ((s_ref[0] + s_ref[1] + g_ref[...]) * dinv_ref[...]
                  + b_ref[...])


def _tc3(s3, gp, dinv, b3):
    return pl.pallas_call(
        _layer3_body,
        grid=(N // _BR,),
        in_specs=[_row_spec(NCLS),
                  pl.BlockSpec((_BR, NCLS), lambda i: (i, 0)),
                  pl.BlockSpec((_BR, 1), lambda i: (i, 0)),
                  pl.BlockSpec((1, NCLS), lambda i: (0, 0))],
        out_specs=pl.BlockSpec((_BR, NCLS), lambda i: (i, 0)),
        out_shape=jax.ShapeDtypeStruct((N, NCLS), jnp.float32),
    )(s3, gp, dinv, b3)


def kernel(nodes, edge_index, emb_weight, W1, b1, W2, b2, W3, b3):
    srcr = edge_index[0].reshape(EROWS, CHUNK)
    dstr = edge_index[1].reshape(EROWS, CHUNK)

    degp = _deg(dstr)
    dinv, xp = _tca(degp, emb_weight)
    s1 = _spmm1(xp, xp, srcr, dstr)
    h1a, h1b = _tc1(s1, xp, dinv, W1, b1.reshape(1, HID))
    s2 = _spmm2(h1a, h1b, srcr, dstr)
    gp = _tc2(s2, h1a, h1b, dinv, W2, b2.reshape(1, HID), W3)
    s3 = _spmm3(gp, srcr, dstr)
    out3 = _tc3(s3, gp, dinv, b3.reshape(1, NCLS))
    return _gather(out3, nodes)
